# Initial kernel scaffold; baseline (speedup 1.0000x reference)
import jax
import jax.numpy as jnp
from jax.experimental import pallas as pl


def kernel(seq_embs, attentions, entity_pos, hts, n_entities, n_rels, W_head, b_head, W_bil, b_bil):
    raise NotImplementedError("write your pallas kernel here")


# speedup vs baseline: 6.1691x; 6.1691x over previous
import jax
import jax.numpy as jnp
from jax.experimental import pallas as pl

EMB = 768
BLK = 64
NC = 97
NKB = EMB // BLK


def _doc_kernel(pos_ref, ht_ref, x_ref, a_ref, wh_ref, bh_ref, wb_ref,
                bb_ref, out_ref):
    f32 = jnp.float32
    pos = pos_ref[0]
    ht = ht_ref[0]
    X = x_ref[0]
    NE, M = pos.shape
    NR = ht.shape[0]
    L = X.shape[0]
    NH = a_ref.shape[1]

    li = jax.lax.broadcasted_iota(jnp.int32, (NE, M, L), 2)
    S = (pos[:, :, None] == li).astype(f32).sum(axis=1)

    EX = jnp.exp(X)
    entT = jnp.log(jax.lax.dot_general(
        EX, S, (((0,), (1,)), ((), ())),
        preferred_element_type=f32))

    ei = jax.lax.broadcasted_iota(jnp.int32, (NR, NE), 1)
    OH = (ht[:, 0][:, None] == ei).astype(f32)
    OT = (ht[:, 1][:, None] == ei).astype(f32)

    hsT_e = jax.lax.dot_general(entT, OH, (((1,), (1,)), ((), ())),
                                preferred_element_type=f32)
    tsT_e = jax.lax.dot_general(entT, OT, (((1,), (1,)), ((), ())),
                                preferred_element_type=f32)

    rsT = jnp.zeros((L, NR), f32)
    for h in range(NH):
        EAhT = jax.lax.dot_general(a_ref[0, h], S, (((0,), (1,)), ((), ())),
                                   preferred_element_type=f32)
        rhT = jax.lax.dot_general(EAhT, OH, (((1,), (1,)), ((), ())),
                                  preferred_element_type=f32)
        rtT = jax.lax.dot_general(EAhT, OT, (((1,), (1,)), ((), ())),
                                  preferred_element_type=f32)
        rsT = rsT + rhT * rtT
    rsT = rsT / jnp.sum(rsT, axis=0, keepdims=True)

    rdocT = jax.lax.dot_general(X, rsT, (((0,), (0,)), ((), ())),
                                preferred_element_type=f32)

    W1 = wh_ref[:, :EMB]
    W2 = wh_ref[:, EMB:]
    b = bh_ref[...]
    hsT = jnp.tanh(
        jax.lax.dot_general(W1, hsT_e, (((1,), (0,)), ((), ())),
                            preferred_element_type=f32)
        + jax.lax.dot_general(W2, rdocT, (((1,), (0,)), ((), ())),
                              preferred_element_type=f32)
        + b)
    tsT = jnp.tanh(
        jax.lax.dot_general(W1, tsT_e, (((1,), (0,)), ((), ())),
                            preferred_element_type=f32)
        + jax.lax.dot_general(W2, rdocT, (((1,), (0,)), ((), ())),
                              preferred_element_type=f32)
        + b)

    acc = jnp.zeros((NC, NR), f32)
    for k in range(NKB):
        hk = hsT[k * BLK:(k + 1) * BLK, :]
        tk = tsT[k * BLK:(k + 1) * BLK, :]
        b3 = hk[:, None, :] * tk[None, :, :]
        b2 = b3.reshape(BLK * BLK, NR)
        acc = acc + jax.lax.dot_general(
            wb_ref[:, k * BLK * BLK:(k + 1) * BLK * BLK], b2,
            (((1,), (0,)), ((), ())),
            preferred_element_type=f32)
    out_ref[0] = acc + bb_ref[...]


def kernel(seq_embs, attentions, entity_pos, hts, n_entities, n_rels,
           W_head, b_head, W_bil, b_bil):
    B, L, Hd = seq_embs.shape
    NH = attentions.shape[1]
    TE = entity_pos.shape[0]
    TR = hts.shape[0]
    NE = TE // B
    M = entity_pos.shape[1]
    NR = TR // B

    pos3 = entity_pos.reshape(B, NE, M)
    hts3 = hts.reshape(B, NR, 2)
    bh = b_head.reshape(EMB, 1)
    bb = b_bil.reshape(NC, 1)

    outT = pl.pallas_call(
        _doc_kernel,
        grid=(B,),
        in_specs=[
            pl.BlockSpec((1, NE, M), lambda i: (i, 0, 0)),
            pl.BlockSpec((1, NR, 2), lambda i: (i, 0, 0)),
            pl.BlockSpec((1, L, Hd), lambda i: (i, 0, 0)),
            pl.BlockSpec((1, NH, L, L), lambda i: (i, 0, 0, 0)),
            pl.BlockSpec((EMB, 2 * Hd), lambda i: (0, 0)),
            pl.BlockSpec((EMB, 1), lambda i: (0, 0)),
            pl.BlockSpec((NC, EMB * BLK), lambda i: (0, 0)),
            pl.BlockSpec((NC, 1), lambda i: (0, 0)),
        ],
        out_specs=pl.BlockSpec((1, NC, NR), lambda i: (i, 0, 0)),
        out_shape=jax.ShapeDtypeStruct((B, NC, NR), jnp.float32),
    )(pos3, hts3, seq_embs, attentions, W_head, bh, W_bil, bb)

    return jnp.transpose(outT, (0, 2, 1)).reshape(TR, NC)


# speedup vs baseline: 6.4216x; 1.0409x over previous
import jax
import jax.numpy as jnp
from jax.experimental import pallas as pl

EMB = 768
BLK = 64
NC = 97
NKB = EMB // BLK


def _doc_kernel(pos_ref, ht_ref, x_ref, a_ref, wh_ref, bh_ref, wb_ref,
                bb_ref, out_ref):
    f32 = jnp.float32
    bf16 = jnp.bfloat16
    pos = pos_ref[0]
    ht = ht_ref[0]
    X = x_ref[0]
    NE, M = pos.shape
    NR = ht.shape[0]
    L = X.shape[0]
    NH = a_ref.shape[1]

    li = jax.lax.broadcasted_iota(jnp.int32, (NE, M, L), 2)
    S = (pos[:, :, None] == li).astype(bf16).sum(axis=1)

    EX = jnp.exp(X).astype(bf16)
    entT = jnp.log(jax.lax.dot_general(
        EX, S, (((0,), (1,)), ((), ())),
        preferred_element_type=f32))

    ei = jax.lax.broadcasted_iota(jnp.int32, (NR, NE), 1)
    OH = (ht[:, 0][:, None] == ei).astype(bf16)
    OT = (ht[:, 1][:, None] == ei).astype(bf16)

    entTb = entT.astype(bf16)
    hsT_e = jax.lax.dot_general(entTb, OH, (((1,), (1,)), ((), ())),
                                preferred_element_type=f32)
    tsT_e = jax.lax.dot_general(entTb, OT, (((1,), (1,)), ((), ())),
                                preferred_element_type=f32)

    rsT = jnp.zeros((L, NR), f32)
    for h in range(NH):
        ah = a_ref[0, h].astype(bf16)
        EAhT = jax.lax.dot_general(ah, S, (((0,), (1,)), ((), ())),
                                   preferred_element_type=f32)
        EAhTb = EAhT.astype(bf16)
        rhT = jax.lax.dot_general(EAhTb, OH, (((1,), (1,)), ((), ())),
                                  preferred_element_type=f32)
        rtT = jax.lax.dot_general(EAhTb, OT, (((1,), (1,)), ((), ())),
                                  preferred_element_type=f32)
        rsT = rsT + rhT * rtT
    rsT = rsT / jnp.sum(rsT, axis=0, keepdims=True)

    rdocT = jax.lax.dot_general(X.astype(bf16), rsT.astype(bf16),
                                (((0,), (0,)), ((), ())),
                                preferred_element_type=f32)

    W1 = wh_ref[:, :EMB].astype(bf16)
    W2 = wh_ref[:, EMB:].astype(bf16)
    b = bh_ref[...]
    ctx = jax.lax.dot_general(W2, rdocT.astype(bf16), (((1,), (0,)), ((), ())),
                              preferred_element_type=f32) + b
    hsT = jnp.tanh(
        jax.lax.dot_general(W1, hsT_e.astype(bf16), (((1,), (0,)), ((), ())),
                            preferred_element_type=f32)
        + ctx).astype(bf16)
    tsT = jnp.tanh(
        jax.lax.dot_general(W1, tsT_e.astype(bf16), (((1,), (0,)), ((), ())),
                            preferred_element_type=f32)
        + ctx).astype(bf16)

    wb = wb_ref[...].astype(bf16)
    acc = jnp.zeros((NC, NR), f32)
    for k in range(NKB):
        hk = hsT[k * BLK:(k + 1) * BLK, :]
        tk = tsT[k * BLK:(k + 1) * BLK, :]
        b3 = hk[:, None, :] * tk[None, :, :]
        b2 = b3.reshape(BLK * BLK, NR)
        acc = acc + jax.lax.dot_general(
            wb[:, k * BLK * BLK:(k + 1) * BLK * BLK], b2,
            (((1,), (0,)), ((), ())),
            preferred_element_type=f32)
    out_ref[0] = acc + bb_ref[...]


def kernel(seq_embs, attentions, entity_pos, hts, n_entities, n_rels,
           W_head, b_head, W_bil, b_bil):
    B, L, Hd = seq_embs.shape
    NH = attentions.shape[1]
    TE = entity_pos.shape[0]
    TR = hts.shape[0]
    NE = TE // B
    M = entity_pos.shape[1]
    NR = TR // B

    pos3 = entity_pos.reshape(B, NE, M)
    hts3 = hts.reshape(B, NR, 2)
    bh = b_head.reshape(EMB, 1)
    bb = b_bil.reshape(NC, 1)

    outT = pl.pallas_call(
        _doc_kernel,
        grid=(B,),
        in_specs=[
            pl.BlockSpec((1, NE, M), lambda i: (i, 0, 0)),
            pl.BlockSpec((1, NR, 2), lambda i: (i, 0, 0)),
            pl.BlockSpec((1, L, Hd), lambda i: (i, 0, 0)),
            pl.BlockSpec((1, NH, L, L), lambda i: (i, 0, 0, 0)),
            pl.BlockSpec((EMB, 2 * Hd), lambda i: (0, 0)),
            pl.BlockSpec((EMB, 1), lambda i: (0, 0)),
            pl.BlockSpec((NC, EMB * BLK), lambda i: (0, 0)),
            pl.BlockSpec((NC, 1), lambda i: (0, 0)),
        ],
        out_specs=pl.BlockSpec((1, NC, NR), lambda i: (i, 0, 0)),
        out_shape=jax.ShapeDtypeStruct((B, NC, NR), jnp.float32),
    )(pos3, hts3, seq_embs, attentions, W_head, bh, W_bil, bb)

    return jnp.transpose(outT, (0, 2, 1)).reshape(TR, NC)


# speedup vs baseline: 7.6520x; 1.1916x over previous
import jax
import jax.numpy as jnp
from jax.experimental import pallas as pl
from jax.experimental.pallas import tpu as pltpu

EMB = 768
BLK = 64
NC = 97
NKB = EMB // BLK
KW = BLK * BLK
NBUF = 2


def _wb_copy(wb_hbm, wb_scr, wb_sem, k):
    return pltpu.make_async_copy(
        wb_hbm.at[:, k * KW:(k + 1) * KW], wb_scr.at[k % NBUF],
        wb_sem.at[k % NBUF])


def _doc_kernel(pos_ref, ht_ref, x_ref, a_ref, wh_ref, bh_ref, bb_ref,
                wb_hbm, out_ref, hs_scr, ts_scr, wb_scr, wb_sem):
    f32 = jnp.float32
    bf16 = jnp.bfloat16
    i = pl.program_id(0)
    B = pl.num_programs(0)
    pos = pos_ref[0]
    ht = ht_ref[0]
    X = x_ref[0]
    NE, M = pos.shape
    NR = ht.shape[0]
    L = X.shape[0]
    NH = a_ref.shape[1]
    NRP = hs_scr.shape[1] // B

    @pl.when(i == 0)
    def _prologue():
        hs_scr[...] = jnp.zeros(hs_scr.shape, bf16)
        ts_scr[...] = jnp.zeros(ts_scr.shape, bf16)
        for k in range(NBUF):
            _wb_copy(wb_hbm, wb_scr, wb_sem, k).start()

    li = jax.lax.broadcasted_iota(jnp.int32, (NE, M, L), 2)
    S = (pos[:, :, None] == li).astype(bf16).sum(axis=1)

    EX = jnp.exp(X).astype(bf16)
    entT = jnp.log(jax.lax.dot_general(
        EX, S, (((0,), (1,)), ((), ())),
        preferred_element_type=f32))

    ei = jax.lax.broadcasted_iota(jnp.int32, (NR, NE), 1)
    OH = (ht[:, 0][:, None] == ei).astype(bf16)
    OT = (ht[:, 1][:, None] == ei).astype(bf16)

    entTb = entT.astype(bf16)
    hsT_e = jax.lax.dot_general(entTb, OH, (((1,), (1,)), ((), ())),
                                preferred_element_type=f32)
    tsT_e = jax.lax.dot_general(entTb, OT, (((1,), (1,)), ((), ())),
                                preferred_element_type=f32)

    Q = jnp.zeros((NE, NE, L), f32)
    for h in range(NH):
        ah = a_ref[0, h].astype(bf16)
        EAh = jax.lax.dot_general(S, ah, (((1,), (0,)), ((), ())),
                                  preferred_element_type=f32)
        EAhb = EAh.astype(bf16)
        Q = Q + EAhb[:, None, :] * EAhb[None, :, :]
    Qr = Q.reshape(NE * NE, L).astype(bf16)

    ci = ht[:, 0] * NE + ht[:, 1]
    qi = jax.lax.broadcasted_iota(jnp.int32, (NR, NE * NE), 1)
    OC = (ci[:, None] == qi).astype(bf16)
    rsT = jax.lax.dot_general(Qr, OC, (((0,), (1,)), ((), ())),
                              preferred_element_type=f32)
    rsT = rsT / jnp.sum(rsT, axis=0, keepdims=True)

    rdocT = jax.lax.dot_general(X.astype(bf16), rsT.astype(bf16),
                                (((0,), (0,)), ((), ())),
                                preferred_element_type=f32)

    W1 = wh_ref[:, :EMB].astype(bf16)
    W2 = wh_ref[:, EMB:].astype(bf16)
    b = bh_ref[...]
    ctx = jax.lax.dot_general(W2, rdocT.astype(bf16), (((1,), (0,)), ((), ())),
                              preferred_element_type=f32) + b
    hsT = jnp.tanh(
        jax.lax.dot_general(W1, hsT_e.astype(bf16), (((1,), (0,)), ((), ())),
                            preferred_element_type=f32)
        + ctx).astype(bf16)
    tsT = jnp.tanh(
        jax.lax.dot_general(W1, tsT_e.astype(bf16), (((1,), (0,)), ((), ())),
                            preferred_element_type=f32)
        + ctx).astype(bf16)

    for d in range(B):
        @pl.when(i == d)
        def _store(d=d, hsT=hsT, tsT=tsT):
            hs_scr[:, d * NRP:d * NRP + NR] = hsT
            ts_scr[:, d * NRP:d * NRP + NR] = tsT

    @pl.when(i == B - 1)
    def _bilinear():
        hsA = hs_scr[...]
        tsA = ts_scr[...]
        NT = hsA.shape[1]
        acc = jnp.zeros((NC, NT), f32)
        for k in range(NKB):
            _wb_copy(wb_hbm, wb_scr, wb_sem, k).wait()
            wbk = wb_scr[k % NBUF].astype(bf16)
            hk = hsA[k * BLK:(k + 1) * BLK, :]
            tk = tsA[k * BLK:(k + 1) * BLK, :]
            b3 = hk[:, None, :] * tk[None, :, :]
            b2 = b3.reshape(KW, NT)
            acc = acc + jax.lax.dot_general(
                wbk, b2, (((1,), (0,)), ((), ())),
                preferred_element_type=f32)
            if k + NBUF < NKB:
                _wb_copy(wb_hbm, wb_scr, wb_sem, k + NBUF).start()
        acc = acc + bb_ref[...]
        for d in range(B):
            out_ref[d] = acc[:, d * NRP:d * NRP + NR]


def kernel(seq_embs, attentions, entity_pos, hts, n_entities, n_rels,
           W_head, b_head, W_bil, b_bil):
    B, L, Hd = seq_embs.shape
    NH = attentions.shape[1]
    TE = entity_pos.shape[0]
    TR = hts.shape[0]
    NE = TE // B
    M = entity_pos.shape[1]
    NR = TR // B
    NRP = ((NR + 127) // 128) * 128

    pos3 = entity_pos.reshape(B, NE, M)
    hts3 = hts.reshape(B, NR, 2)
    bh = b_head.reshape(EMB, 1)
    bb = b_bil.reshape(NC, 1)

    outT = pl.pallas_call(
        _doc_kernel,
        grid=(B,),
        in_specs=[
            pl.BlockSpec((1, NE, M), lambda i: (i, 0, 0)),
            pl.BlockSpec((1, NR, 2), lambda i: (i, 0, 0)),
            pl.BlockSpec((1, L, Hd), lambda i: (i, 0, 0)),
            pl.BlockSpec((1, NH, L, L), lambda i: (i, 0, 0, 0)),
            pl.BlockSpec((EMB, 2 * Hd), lambda i: (0, 0)),
            pl.BlockSpec((EMB, 1), lambda i: (0, 0)),
            pl.BlockSpec((NC, 1), lambda i: (0, 0)),
            pl.BlockSpec(memory_space=pltpu.MemorySpace.HBM),
        ],
        out_specs=pl.BlockSpec((B, NC, NR), lambda i: (0, 0, 0)),
        out_shape=jax.ShapeDtypeStruct((B, NC, NR), jnp.float32),
        scratch_shapes=[
            pltpu.VMEM((EMB, B * NRP), jnp.bfloat16),
            pltpu.VMEM((EMB, B * NRP), jnp.bfloat16),
            pltpu.VMEM((NBUF, NC, KW), jnp.float32),
            pltpu.SemaphoreType.DMA((NBUF,)),
        ],
    )(pos3, hts3, seq_embs, attentions, W_head, bh, bb, W_bil)

    return jnp.transpose(outT, (0, 2, 1)).reshape(TR, NC)
